# baseline (speedup 1.0000x reference)
import functools

import jax
import jax.numpy as jnp
from jax.experimental import pallas as pl
from jax.experimental.pallas import tpu as pltpu

_CT = 8


def _nbrc_body(CT, H, x_ref, h0_ref, u_ref, w_ref, b_ref, y_ref, hf_ref,
               h_s, hb_s, xp_s):
    B = h0_ref.shape[0]
    t0 = pl.program_id(0)

    @pl.when(t0 == 0)
    def _():
        h_s[...] = h0_ref[...]
        hb_s[...] = h0_ref[...].astype(jnp.bfloat16)

    xp_s[...] = (
        jnp.dot(x_ref[...], u_ref[...], preferred_element_type=jnp.float32)
        + b_ref[...]
    )

    h = h_s[...]
    hb = hb_s[...]
    for t in range(CT):
        r = slice(t * B, (t + 1) * B)
        ca = jnp.dot(hb, w_ref[...], preferred_element_type=jnp.float32)
        c = jax.nn.sigmoid(xp_s[r, :H] + ca[:, :H])
        a = 1.0 + jnp.tanh(xp_s[r, H:2 * H] + ca[:, H:])
        hn = c * h + (1.0 - c) * jnp.tanh(xp_s[r, 2 * H:] + a * h)
        y_ref[r, :] = hn
        h = hn
        hb = hn.astype(jnp.bfloat16)
    h_s[...] = h
    hb_s[...] = hb

    @pl.when(t0 == pl.num_programs(0) - 1)
    def _():
        hf_ref[...] = h


def kernel(x_seq, h0, U_c, W_c, b_c, U_a, W_a, b_a, U_h, b_h):
    T, B, I = x_seq.shape
    H = h0.shape[1]
    CT = _CT

    x2 = x_seq.reshape(T * B, I).astype(jnp.bfloat16)
    Ut = jnp.concatenate([U_c.T, U_a.T, U_h.T], axis=1).astype(jnp.bfloat16)
    Wt = jnp.concatenate([W_c.T, W_a.T], axis=1).astype(jnp.bfloat16)
    bb = jnp.concatenate([b_c, b_a, b_h]).reshape(1, 3 * H)

    y2, hf = pl.pallas_call(
        functools.partial(_nbrc_body, CT, H),
        grid=(T // CT,),
        in_specs=[
            pl.BlockSpec((CT * B, I), lambda t: (t, 0)),
            pl.BlockSpec((B, H), lambda t: (0, 0)),
            pl.BlockSpec((I, 3 * H), lambda t: (0, 0)),
            pl.BlockSpec((H, 2 * H), lambda t: (0, 0)),
            pl.BlockSpec((1, 3 * H), lambda t: (0, 0)),
        ],
        out_specs=[
            pl.BlockSpec((CT * B, H), lambda t: (t, 0)),
            pl.BlockSpec((B, H), lambda t: (0, 0)),
        ],
        out_shape=[
            jax.ShapeDtypeStruct((T * B, H), jnp.float32),
            jax.ShapeDtypeStruct((B, H), jnp.float32),
        ],
        scratch_shapes=[
            pltpu.VMEM((B, H), jnp.float32),
            pltpu.VMEM((B, H), jnp.bfloat16),
            pltpu.VMEM((CT * B, 3 * H), jnp.float32),
        ],
        compiler_params=pltpu.CompilerParams(
            dimension_semantics=("arbitrary",),
            vmem_limit_bytes=50 * 1024 * 1024,
        ),
        name="nbrc_scan",
    )(x2, h0, Ut, Wt, bb)

    return (y2.reshape(T, B, H), (hf,))


# speedup vs baseline: 1.1870x; 1.1870x over previous
import functools

import jax
import jax.numpy as jnp
from jax.experimental import pallas as pl
from jax.experimental.pallas import tpu as pltpu

_CT = 4


def _steps(CT, H, B, h, xp_s, w_ref, y_ref, row0):
    for t in range(CT):
        r = slice(t * B, (t + 1) * B)
        ca = jnp.dot(h, w_ref[...], preferred_element_type=jnp.float32)
        c = jax.nn.sigmoid(xp_s[r, :H] + ca[:, :H])
        a = 1.0 + jnp.tanh(xp_s[r, H:2 * H] + ca[:, H:])
        hn = c * h + (1.0 - c) * jnp.tanh(xp_s[r, 2 * H:] + a * h)
        y_ref[row0 + t * B:row0 + (t + 1) * B, :] = hn
        h = hn
    return h


def _nbrc_body(CT, H, x_ref, xn_ref, h0_ref, u_ref, w_ref, b_ref,
               y_ref, hf_ref, h_s, xpA, xpB):
    B = h0_ref.shape[0]
    R = CT * B
    i = pl.program_id(0)

    @pl.when(i == 0)
    def _():
        h_s[...] = h0_ref[...]
        xpA[...] = (
            jnp.dot(x_ref[0:R, :], u_ref[...],
                    preferred_element_type=jnp.float32) + b_ref[...]
        )

    h = h_s[...]
    xpB[...] = (
        jnp.dot(x_ref[R:2 * R, :], u_ref[...],
                preferred_element_type=jnp.float32) + b_ref[...]
    )
    h = _steps(CT, H, B, h, xpA, w_ref, y_ref, 0)
    xpA[...] = (
        jnp.dot(xn_ref[...], u_ref[...],
                preferred_element_type=jnp.float32) + b_ref[...]
    )
    h = _steps(CT, H, B, h, xpB, w_ref, y_ref, R)
    h_s[...] = h

    @pl.when(i == pl.num_programs(0) - 1)
    def _():
        hf_ref[...] = h


def kernel(x_seq, h0, U_c, W_c, b_c, U_a, W_a, b_a, U_h, b_h):
    T, B, I = x_seq.shape
    H = h0.shape[1]
    CT = _CT
    NP = T // (2 * CT)
    NSUB = 2 * NP

    x2 = x_seq.reshape(T * B, I)
    Ut = jnp.concatenate([U_c.T, U_a.T, U_h.T], axis=1)
    Wt = jnp.concatenate([W_c.T, W_a.T], axis=1)
    bb = jnp.concatenate([b_c, b_a, b_h]).reshape(1, 3 * H)

    y2, hf = pl.pallas_call(
        functools.partial(_nbrc_body, CT, H),
        grid=(NP,),
        in_specs=[
            pl.BlockSpec((2 * CT * B, I), lambda i: (i, 0)),
            pl.BlockSpec((CT * B, I),
                         lambda i: (jnp.minimum(2 * i + 2, NSUB - 1), 0)),
            pl.BlockSpec((B, H), lambda i: (0, 0)),
            pl.BlockSpec((I, 3 * H), lambda i: (0, 0)),
            pl.BlockSpec((H, 2 * H), lambda i: (0, 0)),
            pl.BlockSpec((1, 3 * H), lambda i: (0, 0)),
        ],
        out_specs=[
            pl.BlockSpec((2 * CT * B, H), lambda i: (i, 0)),
            pl.BlockSpec((B, H), lambda i: (0, 0)),
        ],
        out_shape=[
            jax.ShapeDtypeStruct((T * B, H), jnp.float32),
            jax.ShapeDtypeStruct((B, H), jnp.float32),
        ],
        scratch_shapes=[
            pltpu.VMEM((B, H), jnp.float32),
            pltpu.VMEM((CT * B, 3 * H), jnp.float32),
            pltpu.VMEM((CT * B, 3 * H), jnp.float32),
        ],
        compiler_params=pltpu.CompilerParams(
            dimension_semantics=("arbitrary",),
            vmem_limit_bytes=50 * 1024 * 1024,
        ),
        name="nbrc_scan",
    )(x2, x2, h0, Ut, Wt, bb)

    return (y2.reshape(T, B, H), (hf,))
